# baseline (speedup 1.0000x reference)
import jax
import jax.numpy as jnp
from jax import lax
from jax.experimental import pallas as pl
from jax.experimental.pallas import tpu as pltpu
from jax.experimental.pallas import tpu_sc as plsc

_NC = 2
_NS = 16
_BN = 1000
_BB = 64


_K = 128


def _make_seg_sum(n_pad, d, e):
    nw = _NC * _NS
    assert e % (nw * _K) == 0 and n_pad % (_NS * _BB) == 0 and d % 128 == 0
    e_per_w = e // nw
    n_chunks = e_per_w // _K
    rpt = n_pad // _NS
    nb = rpt // _BB

    mesh = plsc.VectorSubcoreMesh(core_axis_name="c", subcore_axis_name="s")

    out_type = jax.ShapeDtypeStruct((_NC, n_pad, d), jnp.float32)
    scratch = [
        pltpu.VMEM((_K,), jnp.int32),
        pltpu.VMEM((_K,), jnp.int32),
        pltpu.VMEM((_K, d), jnp.float32),
        pltpu.SemaphoreType.DMA,
        pltpu.VMEM((_K,), jnp.int32),
        pltpu.VMEM((_K,), jnp.int32),
        pltpu.VMEM((_K, d), jnp.float32),
        pltpu.SemaphoreType.DMA,
        pltpu.VMEM((_BB, d), jnp.float32),
        pltpu.VMEM_SHARED((n_pad, d), jnp.float32),
    ]

    def body(h_hbm, src_hbm, dst_hbm, zb_hbm, acc_out,
             s0, d0, r0, g0, s1, d1, r1, g1, fbuf, facc):
        cid = lax.axis_index("c")
        sid = lax.axis_index("s")
        wid = sid * _NC + cid
        bufs = ((s0, d0, r0, g0), (s1, d1, r1, g1))

        pltpu.sync_copy(zb_hbm, fbuf)

        def zero_blk(j, carry):
            blk = pl.ds(sid * rpt + j * _BB, _BB)
            pltpu.sync_copy(fbuf, facc.at[blk])
            return carry

        lax.fori_loop(0, nb, zero_blk, 0)
        plsc.subcore_barrier()

        base0 = wid * e_per_w

        def fetch(j, p):
            sb, db, _, gb = bufs[p]
            base = base0 + j * _K
            pltpu.sync_copy(src_hbm.at[pl.ds(base, _K)], sb)
            pltpu.sync_copy(dst_hbm.at[pl.ds(base, _K)], db)
            pltpu.async_copy(h_hbm.at[sb], bufs[p][2], gb)

        def consume(p):
            sb, db, rb, gb = bufs[p]
            pltpu.make_async_copy(h_hbm.at[sb], rb, gb).wait()
            pltpu.sync_copy(rb, facc.at[db], add=True)

        fetch(0, 0)
        pairs = (n_chunks - 1) // 2

        def pair_step(t, carry):
            j = 2 * t
            fetch(j + 1, 1)
            consume(0)
            fetch(j + 2, 0)
            consume(1)
            return carry

        lax.fori_loop(0, pairs, pair_step, 0)
        for j in range(2 * pairs, n_chunks):
            p = j % 2
            if j + 1 < n_chunks:
                fetch(j + 1, 1 - p)
            consume(p)
        plsc.subcore_barrier()

        def wb_blk(j, carry):
            blk = pl.ds(sid * rpt + j * _BB, _BB)
            pltpu.sync_copy(facc.at[blk], fbuf)
            pltpu.sync_copy(fbuf, acc_out.at[cid, blk])
            return carry

        lax.fori_loop(0, nb, wb_blk, 0)

    return pl.kernel(body, out_type=out_type, mesh=mesh, scratch_types=scratch)


def _make_counts(n_pad, d, e):
    nw = _NC * _NS
    e_per_w = e // nw
    n_chunks = e_per_w // _K
    rpt = n_pad // _NS
    nb = rpt // _BB

    mesh = plsc.VectorSubcoreMesh(core_axis_name="c", subcore_axis_name="s")

    out_type = (jax.ShapeDtypeStruct((_NC, n_pad, d), jnp.float32),
                jax.ShapeDtypeStruct((_NC, n_pad, d), jnp.float32))
    scratch = [
        pltpu.VMEM((_K,), jnp.int32),
        pltpu.SemaphoreType.DMA,
        pltpu.VMEM((_K,), jnp.int32),
        pltpu.SemaphoreType.DMA,
        pltpu.VMEM((_K, d), jnp.float32),
        pltpu.VMEM((_BB, d), jnp.float32),
        pltpu.VMEM_SHARED((n_pad, d), jnp.float32),
    ]

    def body(sd_hbm, td_hbm, zb_hbm, on_hbm, cs_out, ct_out, d0, i0, d1, i1,
             ones_v, fbuf, cacc):
        cid = lax.axis_index("c")
        sid = lax.axis_index("s")
        wid = sid * _NC + cid
        base0 = wid * e_per_w
        bufs = ((d0, i0), (d1, i1))

        pltpu.sync_copy(on_hbm, ones_v)

        for dst_hbm, out in ((sd_hbm, cs_out), (td_hbm, ct_out)):
            pltpu.sync_copy(zb_hbm, fbuf)

            def zero_blk(j, carry):
                blk = pl.ds(sid * rpt + j * _BB, _BB)
                pltpu.sync_copy(fbuf, cacc.at[blk])
                return carry

            lax.fori_loop(0, nb, zero_blk, 0)
            plsc.subcore_barrier()

            def fetch(j, p):
                db, ib = bufs[p]
                base = base0 + j * _K
                pltpu.async_copy(dst_hbm.at[pl.ds(base, _K)], db, ib)

            def consume(j, p):
                db, ib = bufs[p]
                base = base0 + j * _K
                pltpu.make_async_copy(dst_hbm.at[pl.ds(base, _K)], db,
                                      ib).wait()
                pltpu.sync_copy(ones_v, cacc.at[db], add=True)

            fetch(0, 0)
            pairs = (n_chunks - 1) // 2

            def pair_step(t, carry):
                j = 2 * t
                fetch(j + 1, 1)
                consume(j, 0)
                fetch(j + 2, 0)
                consume(j + 1, 1)
                return carry

            lax.fori_loop(0, pairs, pair_step, 0)
            for j in range(2 * pairs, n_chunks):
                p = j % 2
                if j + 1 < n_chunks:
                    fetch(j + 1, 1 - p)
                consume(j, p)
            plsc.subcore_barrier()

            def wb_blk(j, carry):
                blk = pl.ds(sid * rpt + j * _BB, _BB)
                pltpu.sync_copy(cacc.at[blk], fbuf)
                pltpu.sync_copy(fbuf, out.at[cid, blk])
                return carry

            lax.fori_loop(0, nb, wb_blk, 0)
            plsc.subcore_barrier()

    return pl.kernel(body, out_type=out_type, mesh=mesh, scratch_types=scratch)


def _dotT(a, w):
    return lax.dot_general(a, w, (((1,), (1,)), ((), ())),
                           preferred_element_type=jnp.float32,
                           precision=lax.Precision.HIGHEST)


def _tc_layer_pre(acc, cnt, h, wl, wr, bl):
    n, d = h.shape
    grid = n // _BN

    def body(acc_ref, cnt_ref, h_ref, wl_ref, wr_ref, bl_ref, pre_ref, st_ref):
        i = pl.program_id(0)
        s = acc_ref[0] + acc_ref[1]
        c = jnp.sum(cnt_ref[0] + cnt_ref[1], axis=1, keepdims=True) * (1.0 / d)
        inv = 1.0 / jnp.maximum(c, 1.0)
        mean = s * inv
        pre = _dotT(mean, wl_ref[...]) + _dotT(h_ref[...], wr_ref[...]) + bl_ref[...]
        pre_ref[...] = pre
        srow = jnp.sum(pre, axis=0, keepdims=True)
        qrow = jnp.sum(pre * pre, axis=0, keepdims=True)
        upd = jnp.concatenate([srow, qrow, jnp.zeros((6, d), jnp.float32)],
                              axis=0)
        prev = jnp.where(i == 0, jnp.zeros_like(upd), st_ref[...])
        st_ref[...] = prev + upd

    return pl.pallas_call(
        body,
        grid=(grid,),
        in_specs=[
            pl.BlockSpec((_NC, _BN, d), lambda i: (0, i, 0)),
            pl.BlockSpec((_NC, _BN, d), lambda i: (0, i, 0)),
            pl.BlockSpec((_BN, d), lambda i: (i, 0)),
            pl.BlockSpec((d, d), lambda i: (0, 0)),
            pl.BlockSpec((d, d), lambda i: (0, 0)),
            pl.BlockSpec((1, d), lambda i: (0, 0)),
        ],
        out_specs=[
            pl.BlockSpec((_BN, d), lambda i: (i, 0)),
            pl.BlockSpec((8, d), lambda i: (0, 0)),
        ],
        out_shape=[
            jax.ShapeDtypeStruct((n, d), jnp.float32),
            jax.ShapeDtypeStruct((8, d), jnp.float32),
        ],
    )(acc, cnt, h, wl, wr, bl)


def _bn_relu_block(pre_ref, st_ref, g_ref, be_ref, n):
    m = st_ref[0:1, :] * (1.0 / n)
    ex2 = st_ref[1:2, :] * (1.0 / n)
    v = ex2 - m * m
    scale = lax.rsqrt(v + 1e-5) * g_ref[...]
    return jnp.maximum((pre_ref[...] - m) * scale + be_ref[...], 0.0)


def _tc_bn_relu(pre, st, g, be):
    n, d = pre.shape
    grid = n // _BN

    def body(pre_ref, st_ref, g_ref, be_ref, out_ref):
        out_ref[...] = _bn_relu_block(pre_ref, st_ref, g_ref, be_ref, n)

    return pl.pallas_call(
        body,
        grid=(grid,),
        in_specs=[
            pl.BlockSpec((_BN, d), lambda i: (i, 0)),
            pl.BlockSpec((8, d), lambda i: (0, 0)),
            pl.BlockSpec((1, d), lambda i: (0, 0)),
            pl.BlockSpec((1, d), lambda i: (0, 0)),
        ],
        out_specs=pl.BlockSpec((_BN, d), lambda i: (i, 0)),
        out_shape=jax.ShapeDtypeStruct((n, d), jnp.float32),
    )(pre, st, g, be)


def _tc_bn_relu_fin(pre, st, g, be, wf, bf):
    n, d = pre.shape
    grid = n // _BN

    def body(pre_ref, st_ref, g_ref, be_ref, wf_ref, bf_ref, out_ref):
        hblk = _bn_relu_block(pre_ref, st_ref, g_ref, be_ref, n)
        out_ref[...] = _dotT(hblk, wf_ref[...]) + bf_ref[...]

    return pl.pallas_call(
        body,
        grid=(grid,),
        in_specs=[
            pl.BlockSpec((_BN, d), lambda i: (i, 0)),
            pl.BlockSpec((8, d), lambda i: (0, 0)),
            pl.BlockSpec((1, d), lambda i: (0, 0)),
            pl.BlockSpec((1, d), lambda i: (0, 0)),
            pl.BlockSpec((d, d), lambda i: (0, 0)),
            pl.BlockSpec((1, d), lambda i: (0, 0)),
        ],
        out_specs=pl.BlockSpec((_BN, d), lambda i: (i, 0)),
        out_shape=jax.ShapeDtypeStruct((n, d), jnp.float32),
    )(pre, st, g, be, wf, bf)


def kernel(x, edge_index_spatial, edge_index_temporal,
           Wl0, bl0, Wr0, g0, be0,
           Wl1, bl1, Wr1, g1, be1,
           Wl2, bl2, Wr2, g2, be2,
           Wl3, bl3, Wr3, g3, be3,
           Wfin, bfin):
    n, d = x.shape
    e = edge_index_spatial.shape[1]
    n_pad = -(-n // (_NS * _BB)) * _NS * _BB
    e_pad = -(-e // (_NC * _NS * _K)) * _NC * _NS * _K
    pz = jnp.zeros((e_pad - e,), jnp.int32)
    pn = jnp.full((e_pad - e,), n, jnp.int32)
    ss, sd = edge_index_spatial[0], edge_index_spatial[1]
    ts, td = edge_index_temporal[0], edge_index_temporal[1]
    ss, sd = jnp.concatenate([ss, pz]), jnp.concatenate([sd, pn])
    ts, td = jnp.concatenate([ts, pz]), jnp.concatenate([td, pn])
    r1 = lambda v: jnp.reshape(v, (1, d))
    zb = jnp.zeros((_BB, d), jnp.float32)
    on = jnp.ones((_K, d), jnp.float32)

    seg = _make_seg_sum(n_pad, d, e_pad)
    cnt_s, cnt_t = _make_counts(n_pad, d, e_pad)(sd, td, zb, on)

    acc = seg(x, ss, sd, zb)
    pre, st = _tc_layer_pre(acc, cnt_s, x, Wl0, Wr0, r1(bl0))
    h = _tc_bn_relu(pre, st, r1(g0), r1(be0))

    acc = seg(h, ss, sd, zb)
    pre, st = _tc_layer_pre(acc, cnt_s, h, Wl1, Wr1, r1(bl1))
    h = _tc_bn_relu(pre, st, r1(g1), r1(be1))

    acc = seg(h, ts, td, zb)
    pre, st = _tc_layer_pre(acc, cnt_t, h, Wl2, Wr2, r1(bl2))
    h = _tc_bn_relu(pre, st, r1(g2), r1(be2))

    acc = seg(h, ts, td, zb)
    pre, st = _tc_layer_pre(acc, cnt_t, h, Wl3, Wr3, r1(bl3))
    return _tc_bn_relu_fin(pre, st, r1(g3), r1(be3), Wfin, r1(bfin))


# speedup vs baseline: 1.0827x; 1.0827x over previous
import jax
import jax.numpy as jnp
from jax import lax
from jax.experimental import pallas as pl
from jax.experimental.pallas import tpu as pltpu
from jax.experimental.pallas import tpu_sc as plsc

_NC = 2
_NS = 16
_BN = 1000
_BB = 64


_K = 128
_Q1_FRAC = 0.35


def _make_seg_sum(n_pad, d, e):
    nw = _NC * _NS
    assert e % (nw * _K) == 0 and n_pad % (_NS * _BB) == 0 and d % 128 == 0
    chunks_per_pair = e // (_NS * _K)
    q1 = max(1, round(chunks_per_pair * _Q1_FRAC))
    q0 = chunks_per_pair - q1
    rpt = n_pad // _NS
    nb = rpt // _BB

    mesh = plsc.VectorSubcoreMesh(core_axis_name="c", subcore_axis_name="s")

    out_type = jax.ShapeDtypeStruct((_NC, n_pad, d), jnp.float32)
    scratch = [
        pltpu.VMEM((_K,), jnp.int32),
        pltpu.VMEM((_K,), jnp.int32),
        pltpu.VMEM((_K, d), jnp.float32),
        pltpu.SemaphoreType.DMA,
        pltpu.VMEM((_K,), jnp.int32),
        pltpu.VMEM((_K,), jnp.int32),
        pltpu.VMEM((_K, d), jnp.float32),
        pltpu.SemaphoreType.DMA,
        pltpu.VMEM((_BB, d), jnp.float32),
        pltpu.VMEM_SHARED((n_pad, d), jnp.float32),
    ]

    def body(h_hbm, src_hbm, dst_hbm, zb_hbm, acc_out,
             s0, d0, r0, g0, s1, d1, r1, g1, fbuf, facc):
        cid = lax.axis_index("c")
        sid = lax.axis_index("s")
        wid = sid * _NC + cid
        bufs = ((s0, d0, r0, g0), (s1, d1, r1, g1))

        pltpu.sync_copy(zb_hbm, fbuf)

        def zero_blk(j, carry):
            blk = pl.ds(sid * rpt + j * _BB, _BB)
            pltpu.sync_copy(fbuf, facc.at[blk])
            return carry

        lax.fori_loop(0, nb, zero_blk, 0)
        plsc.subcore_barrier()

        def fetch(base0, j, p):
            sb, db, _, gb = bufs[p]
            base = base0 + j * _K
            pltpu.sync_copy(src_hbm.at[pl.ds(base, _K)], sb)
            pltpu.sync_copy(dst_hbm.at[pl.ds(base, _K)], db)
            pltpu.async_copy(h_hbm.at[sb], bufs[p][2], gb)

        def consume(p):
            sb, db, rb, gb = bufs[p]
            pltpu.make_async_copy(h_hbm.at[sb], rb, gb).wait()
            pltpu.sync_copy(rb, facc.at[db], add=True)

        def run_pipe(base0, n_chunks):
            fetch(base0, 0, 0)
            pairs = (n_chunks - 1) // 2

            def pair_step(t, carry):
                j = 2 * t
                fetch(base0, j + 1, 1)
                consume(0)
                fetch(base0, j + 2, 0)
                consume(1)
                return carry

            lax.fori_loop(0, pairs, pair_step, 0)
            for j in range(2 * pairs, n_chunks):
                p = j % 2
                if j + 1 < n_chunks:
                    fetch(base0, j + 1, 1 - p)
                consume(p)

        @pl.when(cid == 0)
        def _():
            run_pipe(sid * q0 * _K, q0)

        @pl.when(cid == 1)
        def _():
            run_pipe((_NS * q0 + sid * q1) * _K, q1)

        plsc.subcore_barrier()

        def wb_blk(j, carry):
            blk = pl.ds(sid * rpt + j * _BB, _BB)
            pltpu.sync_copy(facc.at[blk], fbuf)
            pltpu.sync_copy(fbuf, acc_out.at[cid, blk])
            return carry

        lax.fori_loop(0, nb, wb_blk, 0)

    return pl.kernel(body, out_type=out_type, mesh=mesh, scratch_types=scratch)


def _make_counts(n_pad, d, e):
    nw = _NC * _NS
    e_per_w = e // nw
    n_chunks = e_per_w // _K
    rpt = n_pad // _NS
    nb = rpt // _BB

    mesh = plsc.VectorSubcoreMesh(core_axis_name="c", subcore_axis_name="s")

    out_type = (jax.ShapeDtypeStruct((_NC, n_pad, d), jnp.float32),
                jax.ShapeDtypeStruct((_NC, n_pad, d), jnp.float32))
    scratch = [
        pltpu.VMEM((_K,), jnp.int32),
        pltpu.SemaphoreType.DMA,
        pltpu.VMEM((_K,), jnp.int32),
        pltpu.SemaphoreType.DMA,
        pltpu.VMEM((_K, d), jnp.float32),
        pltpu.VMEM((_BB, d), jnp.float32),
        pltpu.VMEM_SHARED((n_pad, d), jnp.float32),
    ]

    def body(sd_hbm, td_hbm, zb_hbm, on_hbm, cs_out, ct_out, d0, i0, d1, i1,
             ones_v, fbuf, cacc):
        cid = lax.axis_index("c")
        sid = lax.axis_index("s")
        wid = sid * _NC + cid
        base0 = wid * e_per_w
        bufs = ((d0, i0), (d1, i1))

        pltpu.sync_copy(on_hbm, ones_v)

        for dst_hbm, out in ((sd_hbm, cs_out), (td_hbm, ct_out)):
            pltpu.sync_copy(zb_hbm, fbuf)

            def zero_blk(j, carry):
                blk = pl.ds(sid * rpt + j * _BB, _BB)
                pltpu.sync_copy(fbuf, cacc.at[blk])
                return carry

            lax.fori_loop(0, nb, zero_blk, 0)
            plsc.subcore_barrier()

            def fetch(j, p):
                db, ib = bufs[p]
                base = base0 + j * _K
                pltpu.async_copy(dst_hbm.at[pl.ds(base, _K)], db, ib)

            def consume(j, p):
                db, ib = bufs[p]
                base = base0 + j * _K
                pltpu.make_async_copy(dst_hbm.at[pl.ds(base, _K)], db,
                                      ib).wait()
                pltpu.sync_copy(ones_v, cacc.at[db], add=True)

            fetch(0, 0)
            pairs = (n_chunks - 1) // 2

            def pair_step(t, carry):
                j = 2 * t
                fetch(j + 1, 1)
                consume(j, 0)
                fetch(j + 2, 0)
                consume(j + 1, 1)
                return carry

            lax.fori_loop(0, pairs, pair_step, 0)
            for j in range(2 * pairs, n_chunks):
                p = j % 2
                if j + 1 < n_chunks:
                    fetch(j + 1, 1 - p)
                consume(j, p)
            plsc.subcore_barrier()

            def wb_blk(j, carry):
                blk = pl.ds(sid * rpt + j * _BB, _BB)
                pltpu.sync_copy(cacc.at[blk], fbuf)
                pltpu.sync_copy(fbuf, out.at[cid, blk])
                return carry

            lax.fori_loop(0, nb, wb_blk, 0)
            plsc.subcore_barrier()

    return pl.kernel(body, out_type=out_type, mesh=mesh, scratch_types=scratch)


def _dotT(a, w):
    return lax.dot_general(a, w, (((1,), (1,)), ((), ())),
                           preferred_element_type=jnp.float32,
                           precision=lax.Precision.HIGHEST)


def _tc_layer_pre(acc, cnt, h, wl, wr, bl):
    n, d = h.shape
    grid = n // _BN

    def body(acc_ref, cnt_ref, h_ref, wl_ref, wr_ref, bl_ref, pre_ref, st_ref):
        i = pl.program_id(0)
        s = acc_ref[0] + acc_ref[1]
        c = jnp.sum(cnt_ref[0] + cnt_ref[1], axis=1, keepdims=True) * (1.0 / d)
        inv = 1.0 / jnp.maximum(c, 1.0)
        mean = s * inv
        pre = _dotT(mean, wl_ref[...]) + _dotT(h_ref[...], wr_ref[...]) + bl_ref[...]
        pre_ref[...] = pre
        srow = jnp.sum(pre, axis=0, keepdims=True)
        qrow = jnp.sum(pre * pre, axis=0, keepdims=True)
        upd = jnp.concatenate([srow, qrow, jnp.zeros((6, d), jnp.float32)],
                              axis=0)
        prev = jnp.where(i == 0, jnp.zeros_like(upd), st_ref[...])
        st_ref[...] = prev + upd

    return pl.pallas_call(
        body,
        grid=(grid,),
        in_specs=[
            pl.BlockSpec((_NC, _BN, d), lambda i: (0, i, 0)),
            pl.BlockSpec((_NC, _BN, d), lambda i: (0, i, 0)),
            pl.BlockSpec((_BN, d), lambda i: (i, 0)),
            pl.BlockSpec((d, d), lambda i: (0, 0)),
            pl.BlockSpec((d, d), lambda i: (0, 0)),
            pl.BlockSpec((1, d), lambda i: (0, 0)),
        ],
        out_specs=[
            pl.BlockSpec((_BN, d), lambda i: (i, 0)),
            pl.BlockSpec((8, d), lambda i: (0, 0)),
        ],
        out_shape=[
            jax.ShapeDtypeStruct((n, d), jnp.float32),
            jax.ShapeDtypeStruct((8, d), jnp.float32),
        ],
    )(acc, cnt, h, wl, wr, bl)


def _bn_relu_block(pre_ref, st_ref, g_ref, be_ref, n):
    m = st_ref[0:1, :] * (1.0 / n)
    ex2 = st_ref[1:2, :] * (1.0 / n)
    v = ex2 - m * m
    scale = lax.rsqrt(v + 1e-5) * g_ref[...]
    return jnp.maximum((pre_ref[...] - m) * scale + be_ref[...], 0.0)


def _tc_bn_relu(pre, st, g, be):
    n, d = pre.shape
    grid = n // _BN

    def body(pre_ref, st_ref, g_ref, be_ref, out_ref):
        out_ref[...] = _bn_relu_block(pre_ref, st_ref, g_ref, be_ref, n)

    return pl.pallas_call(
        body,
        grid=(grid,),
        in_specs=[
            pl.BlockSpec((_BN, d), lambda i: (i, 0)),
            pl.BlockSpec((8, d), lambda i: (0, 0)),
            pl.BlockSpec((1, d), lambda i: (0, 0)),
            pl.BlockSpec((1, d), lambda i: (0, 0)),
        ],
        out_specs=pl.BlockSpec((_BN, d), lambda i: (i, 0)),
        out_shape=jax.ShapeDtypeStruct((n, d), jnp.float32),
    )(pre, st, g, be)


def _tc_bn_relu_fin(pre, st, g, be, wf, bf):
    n, d = pre.shape
    grid = n // _BN

    def body(pre_ref, st_ref, g_ref, be_ref, wf_ref, bf_ref, out_ref):
        hblk = _bn_relu_block(pre_ref, st_ref, g_ref, be_ref, n)
        out_ref[...] = _dotT(hblk, wf_ref[...]) + bf_ref[...]

    return pl.pallas_call(
        body,
        grid=(grid,),
        in_specs=[
            pl.BlockSpec((_BN, d), lambda i: (i, 0)),
            pl.BlockSpec((8, d), lambda i: (0, 0)),
            pl.BlockSpec((1, d), lambda i: (0, 0)),
            pl.BlockSpec((1, d), lambda i: (0, 0)),
            pl.BlockSpec((d, d), lambda i: (0, 0)),
            pl.BlockSpec((1, d), lambda i: (0, 0)),
        ],
        out_specs=pl.BlockSpec((_BN, d), lambda i: (i, 0)),
        out_shape=jax.ShapeDtypeStruct((n, d), jnp.float32),
    )(pre, st, g, be, wf, bf)


def kernel(x, edge_index_spatial, edge_index_temporal,
           Wl0, bl0, Wr0, g0, be0,
           Wl1, bl1, Wr1, g1, be1,
           Wl2, bl2, Wr2, g2, be2,
           Wl3, bl3, Wr3, g3, be3,
           Wfin, bfin):
    n, d = x.shape
    e = edge_index_spatial.shape[1]
    n_pad = -(-n // (_NS * _BB)) * _NS * _BB
    e_pad = -(-e // (_NC * _NS * _K)) * _NC * _NS * _K
    pz = jnp.zeros((e_pad - e,), jnp.int32)
    pn = jnp.full((e_pad - e,), n, jnp.int32)
    ss, sd = edge_index_spatial[0], edge_index_spatial[1]
    ts, td = edge_index_temporal[0], edge_index_temporal[1]
    ss, sd = jnp.concatenate([ss, pz]), jnp.concatenate([sd, pn])
    ts, td = jnp.concatenate([ts, pz]), jnp.concatenate([td, pn])
    r1 = lambda v: jnp.reshape(v, (1, d))
    zb = jnp.zeros((_BB, d), jnp.float32)
    on = jnp.ones((_K, d), jnp.float32)

    seg = _make_seg_sum(n_pad, d, e_pad)
    cnt_s, cnt_t = _make_counts(n_pad, d, e_pad)(sd, td, zb, on)

    acc = seg(x, ss, sd, zb)
    pre, st = _tc_layer_pre(acc, cnt_s, x, Wl0, Wr0, r1(bl0))
    h = _tc_bn_relu(pre, st, r1(g0), r1(be0))

    acc = seg(h, ss, sd, zb)
    pre, st = _tc_layer_pre(acc, cnt_s, h, Wl1, Wr1, r1(bl1))
    h = _tc_bn_relu(pre, st, r1(g1), r1(be1))

    acc = seg(h, ts, td, zb)
    pre, st = _tc_layer_pre(acc, cnt_t, h, Wl2, Wr2, r1(bl2))
    h = _tc_bn_relu(pre, st, r1(g2), r1(be2))

    acc = seg(h, ts, td, zb)
    pre, st = _tc_layer_pre(acc, cnt_t, h, Wl3, Wr3, r1(bl3))
    return _tc_bn_relu_fin(pre, st, r1(g3), r1(be3), Wfin, r1(bfin))


# speedup vs baseline: 1.1543x; 1.0662x over previous
import jax
import jax.numpy as jnp
from jax import lax
from jax.experimental import pallas as pl
from jax.experimental.pallas import tpu as pltpu
from jax.experimental.pallas import tpu_sc as plsc

_NC = 2
_NS = 16
_BN = 1000
_BB = 64


_K = 128
_Q1_FRAC = 0.23


def _make_seg_sum(n_pad, d, e):
    nw = _NC * _NS
    assert e % (nw * _K) == 0 and n_pad % (_NS * _BB) == 0 and d % 128 == 0
    chunks_per_pair = e // (_NS * _K)
    q1 = max(1, round(chunks_per_pair * _Q1_FRAC))
    q0 = chunks_per_pair - q1
    rpt = n_pad // _NS
    nb = rpt // _BB

    mesh = plsc.VectorSubcoreMesh(core_axis_name="c", subcore_axis_name="s")

    out_type = jax.ShapeDtypeStruct((_NC, n_pad, d), jnp.float32)
    scratch = [
        pltpu.VMEM((_K,), jnp.int32),
        pltpu.VMEM((_K,), jnp.int32),
        pltpu.VMEM((_K, d), jnp.float32),
        pltpu.SemaphoreType.DMA,
        pltpu.VMEM((_K,), jnp.int32),
        pltpu.VMEM((_K,), jnp.int32),
        pltpu.VMEM((_K, d), jnp.float32),
        pltpu.SemaphoreType.DMA,
        pltpu.VMEM((_BB, d), jnp.float32),
        pltpu.VMEM_SHARED((n_pad, d), jnp.float32),
    ]

    def body(h_hbm, src_hbm, dst_hbm, zb_hbm, acc_out,
             s0, d0, r0, g0, s1, d1, r1, g1, fbuf, facc):
        cid = lax.axis_index("c")
        sid = lax.axis_index("s")
        wid = sid * _NC + cid
        bufs = ((s0, d0, r0, g0), (s1, d1, r1, g1))

        pltpu.sync_copy(zb_hbm, fbuf)

        def zero_blk(j, carry):
            blk = pl.ds(sid * rpt + j * _BB, _BB)
            pltpu.sync_copy(fbuf, facc.at[blk])
            return carry

        lax.fori_loop(0, nb, zero_blk, 0)
        plsc.subcore_barrier()

        def fetch(base0, j, p):
            sb, db, _, gb = bufs[p]
            base = base0 + j * _K
            pltpu.sync_copy(src_hbm.at[pl.ds(base, _K)], sb)
            pltpu.sync_copy(dst_hbm.at[pl.ds(base, _K)], db)
            pltpu.async_copy(h_hbm.at[sb], bufs[p][2], gb)

        def consume(p):
            sb, db, rb, gb = bufs[p]
            pltpu.make_async_copy(h_hbm.at[sb], rb, gb).wait()
            pltpu.sync_copy(rb, facc.at[db], add=True)

        def run_pipe(base0, n_chunks):
            fetch(base0, 0, 0)
            pairs = (n_chunks - 1) // 2

            def pair_step(t, carry):
                j = 2 * t
                fetch(base0, j + 1, 1)
                consume(0)
                fetch(base0, j + 2, 0)
                consume(1)
                return carry

            lax.fori_loop(0, pairs, pair_step, 0)
            for j in range(2 * pairs, n_chunks):
                p = j % 2
                if j + 1 < n_chunks:
                    fetch(base0, j + 1, 1 - p)
                consume(p)

        @pl.when(cid == 0)
        def _():
            run_pipe(sid * q0 * _K, q0)

        @pl.when(cid == 1)
        def _():
            run_pipe((_NS * q0 + sid * q1) * _K, q1)

        plsc.subcore_barrier()

        def wb_blk(j, carry):
            blk = pl.ds(sid * rpt + j * _BB, _BB)
            pltpu.sync_copy(facc.at[blk], fbuf)
            pltpu.sync_copy(fbuf, acc_out.at[cid, blk])
            return carry

        lax.fori_loop(0, nb, wb_blk, 0)

    return pl.kernel(body, out_type=out_type, mesh=mesh, scratch_types=scratch)


def _make_counts(n_pad, d, e):
    nw = _NC * _NS
    e_per_w = e // nw
    n_chunks = e_per_w // _K
    rpt = n_pad // _NS
    nb = rpt // _BB

    mesh = plsc.VectorSubcoreMesh(core_axis_name="c", subcore_axis_name="s")

    out_type = (jax.ShapeDtypeStruct((_NC, n_pad, d), jnp.float32),
                jax.ShapeDtypeStruct((_NC, n_pad, d), jnp.float32))
    scratch = [
        pltpu.VMEM((_K,), jnp.int32),
        pltpu.SemaphoreType.DMA,
        pltpu.VMEM((_K,), jnp.int32),
        pltpu.SemaphoreType.DMA,
        pltpu.VMEM((_K, d), jnp.float32),
        pltpu.VMEM((_BB, d), jnp.float32),
        pltpu.VMEM_SHARED((n_pad, d), jnp.float32),
    ]

    def body(sd_hbm, td_hbm, zb_hbm, on_hbm, cs_out, ct_out, d0, i0, d1, i1,
             ones_v, fbuf, cacc):
        cid = lax.axis_index("c")
        sid = lax.axis_index("s")
        wid = sid * _NC + cid
        base0 = wid * e_per_w
        bufs = ((d0, i0), (d1, i1))

        pltpu.sync_copy(on_hbm, ones_v)

        for dst_hbm, out in ((sd_hbm, cs_out), (td_hbm, ct_out)):
            pltpu.sync_copy(zb_hbm, fbuf)

            def zero_blk(j, carry):
                blk = pl.ds(sid * rpt + j * _BB, _BB)
                pltpu.sync_copy(fbuf, cacc.at[blk])
                return carry

            lax.fori_loop(0, nb, zero_blk, 0)
            plsc.subcore_barrier()

            def fetch(j, p):
                db, ib = bufs[p]
                base = base0 + j * _K
                pltpu.async_copy(dst_hbm.at[pl.ds(base, _K)], db, ib)

            def consume(j, p):
                db, ib = bufs[p]
                base = base0 + j * _K
                pltpu.make_async_copy(dst_hbm.at[pl.ds(base, _K)], db,
                                      ib).wait()
                pltpu.sync_copy(ones_v, cacc.at[db], add=True)

            fetch(0, 0)
            pairs = (n_chunks - 1) // 2

            def pair_step(t, carry):
                j = 2 * t
                fetch(j + 1, 1)
                consume(j, 0)
                fetch(j + 2, 0)
                consume(j + 1, 1)
                return carry

            lax.fori_loop(0, pairs, pair_step, 0)
            for j in range(2 * pairs, n_chunks):
                p = j % 2
                if j + 1 < n_chunks:
                    fetch(j + 1, 1 - p)
                consume(j, p)
            plsc.subcore_barrier()

            def wb_blk(j, carry):
                blk = pl.ds(sid * rpt + j * _BB, _BB)
                pltpu.sync_copy(cacc.at[blk], fbuf)
                pltpu.sync_copy(fbuf, out.at[cid, blk])
                return carry

            lax.fori_loop(0, nb, wb_blk, 0)
            plsc.subcore_barrier()

    return pl.kernel(body, out_type=out_type, mesh=mesh, scratch_types=scratch)


def _dotT(a, w):
    return lax.dot_general(a, w, (((1,), (1,)), ((), ())),
                           preferred_element_type=jnp.float32,
                           precision=lax.Precision.HIGHEST)


def _tc_layer_pre(acc, cnt, h, wl, wr, bl):
    n, d = h.shape
    grid = n // _BN

    def body(acc_ref, cnt_ref, h_ref, wl_ref, wr_ref, bl_ref, pre_ref, st_ref):
        i = pl.program_id(0)
        s = acc_ref[0] + acc_ref[1]
        c = jnp.sum(cnt_ref[0] + cnt_ref[1], axis=1, keepdims=True) * (1.0 / d)
        inv = 1.0 / jnp.maximum(c, 1.0)
        mean = s * inv
        pre = _dotT(mean, wl_ref[...]) + _dotT(h_ref[...], wr_ref[...]) + bl_ref[...]
        pre_ref[...] = pre
        srow = jnp.sum(pre, axis=0, keepdims=True)
        qrow = jnp.sum(pre * pre, axis=0, keepdims=True)
        upd = jnp.concatenate([srow, qrow, jnp.zeros((6, d), jnp.float32)],
                              axis=0)
        prev = jnp.where(i == 0, jnp.zeros_like(upd), st_ref[...])
        st_ref[...] = prev + upd

    return pl.pallas_call(
        body,
        grid=(grid,),
        in_specs=[
            pl.BlockSpec((_NC, _BN, d), lambda i: (0, i, 0)),
            pl.BlockSpec((_NC, _BN, d), lambda i: (0, i, 0)),
            pl.BlockSpec((_BN, d), lambda i: (i, 0)),
            pl.BlockSpec((d, d), lambda i: (0, 0)),
            pl.BlockSpec((d, d), lambda i: (0, 0)),
            pl.BlockSpec((1, d), lambda i: (0, 0)),
        ],
        out_specs=[
            pl.BlockSpec((_BN, d), lambda i: (i, 0)),
            pl.BlockSpec((8, d), lambda i: (0, 0)),
        ],
        out_shape=[
            jax.ShapeDtypeStruct((n, d), jnp.float32),
            jax.ShapeDtypeStruct((8, d), jnp.float32),
        ],
    )(acc, cnt, h, wl, wr, bl)


def _bn_relu_block(pre_ref, st_ref, g_ref, be_ref, n):
    m = st_ref[0:1, :] * (1.0 / n)
    ex2 = st_ref[1:2, :] * (1.0 / n)
    v = ex2 - m * m
    scale = lax.rsqrt(v + 1e-5) * g_ref[...]
    return jnp.maximum((pre_ref[...] - m) * scale + be_ref[...], 0.0)


def _tc_bn_relu(pre, st, g, be):
    n, d = pre.shape
    grid = n // _BN

    def body(pre_ref, st_ref, g_ref, be_ref, out_ref):
        out_ref[...] = _bn_relu_block(pre_ref, st_ref, g_ref, be_ref, n)

    return pl.pallas_call(
        body,
        grid=(grid,),
        in_specs=[
            pl.BlockSpec((_BN, d), lambda i: (i, 0)),
            pl.BlockSpec((8, d), lambda i: (0, 0)),
            pl.BlockSpec((1, d), lambda i: (0, 0)),
            pl.BlockSpec((1, d), lambda i: (0, 0)),
        ],
        out_specs=pl.BlockSpec((_BN, d), lambda i: (i, 0)),
        out_shape=jax.ShapeDtypeStruct((n, d), jnp.float32),
    )(pre, st, g, be)


def _tc_bn_relu_fin(pre, st, g, be, wf, bf):
    n, d = pre.shape
    grid = n // _BN

    def body(pre_ref, st_ref, g_ref, be_ref, wf_ref, bf_ref, out_ref):
        hblk = _bn_relu_block(pre_ref, st_ref, g_ref, be_ref, n)
        out_ref[...] = _dotT(hblk, wf_ref[...]) + bf_ref[...]

    return pl.pallas_call(
        body,
        grid=(grid,),
        in_specs=[
            pl.BlockSpec((_BN, d), lambda i: (i, 0)),
            pl.BlockSpec((8, d), lambda i: (0, 0)),
            pl.BlockSpec((1, d), lambda i: (0, 0)),
            pl.BlockSpec((1, d), lambda i: (0, 0)),
            pl.BlockSpec((d, d), lambda i: (0, 0)),
            pl.BlockSpec((1, d), lambda i: (0, 0)),
        ],
        out_specs=pl.BlockSpec((_BN, d), lambda i: (i, 0)),
        out_shape=jax.ShapeDtypeStruct((n, d), jnp.float32),
    )(pre, st, g, be, wf, bf)


def kernel(x, edge_index_spatial, edge_index_temporal,
           Wl0, bl0, Wr0, g0, be0,
           Wl1, bl1, Wr1, g1, be1,
           Wl2, bl2, Wr2, g2, be2,
           Wl3, bl3, Wr3, g3, be3,
           Wfin, bfin):
    n, d = x.shape
    e = edge_index_spatial.shape[1]
    n_pad = -(-n // (_NS * _BB)) * _NS * _BB
    e_pad = -(-e // (_NC * _NS * _K)) * _NC * _NS * _K
    pz = jnp.zeros((e_pad - e,), jnp.int32)
    pn = jnp.full((e_pad - e,), n, jnp.int32)
    ss, sd = edge_index_spatial[0], edge_index_spatial[1]
    ts, td = edge_index_temporal[0], edge_index_temporal[1]
    ss, sd = jnp.concatenate([ss, pz]), jnp.concatenate([sd, pn])
    ts, td = jnp.concatenate([ts, pz]), jnp.concatenate([td, pn])
    r1 = lambda v: jnp.reshape(v, (1, d))
    zb = jnp.zeros((_BB, d), jnp.float32)
    on = jnp.ones((_K, d), jnp.float32)

    seg = _make_seg_sum(n_pad, d, e_pad)
    cnt_s, cnt_t = _make_counts(n_pad, d, e_pad)(sd, td, zb, on)

    acc = seg(x, ss, sd, zb)
    pre, st = _tc_layer_pre(acc, cnt_s, x, Wl0, Wr0, r1(bl0))
    h = _tc_bn_relu(pre, st, r1(g0), r1(be0))

    acc = seg(h, ss, sd, zb)
    pre, st = _tc_layer_pre(acc, cnt_s, h, Wl1, Wr1, r1(bl1))
    h = _tc_bn_relu(pre, st, r1(g1), r1(be1))

    acc = seg(h, ts, td, zb)
    pre, st = _tc_layer_pre(acc, cnt_t, h, Wl2, Wr2, r1(bl2))
    h = _tc_bn_relu(pre, st, r1(g2), r1(be2))

    acc = seg(h, ts, td, zb)
    pre, st = _tc_layer_pre(acc, cnt_t, h, Wl3, Wr3, r1(bl3))
    return _tc_bn_relu_fin(pre, st, r1(g3), r1(be3), Wfin, r1(bfin))
